# baseline (speedup 1.0000x reference)
import functools

import jax
import jax.numpy as jnp
from jax import lax
from jax.experimental import pallas as pl
from jax.experimental.pallas import tpu as pltpu
from jax.experimental.pallas import tpu_sc as plsc

E = 8
TOPK = 2
DIM = 1024
INTER = 512
ROUTE_SCALE = 2.5
T = 2048

BLK = 256
NBLK = T * TOPK // BLK + E
R = NBLK * BLK
NBLK2 = NBLK + T // BLK
R2 = NBLK2 * BLK

NTILE = 16
NW = 32
TT = T // NTILE
_MESH = plsc.VectorSubcoreMesh(core_axis_name="c", subcore_axis_name="s",
                               num_cores=2, num_subcores=NTILE)


def _scores_body(x_ref, gw_ref, eb_ref, sc_ref, bi_ref, x16_ref):
    st = lax.dot_general(gw_ref[...], x_ref[...], (((1,), (1,)), ((), ())),
                         preferred_element_type=jnp.float32)
    s = jax.nn.sigmoid(st)
    sc_ref[...] = s
    bi_ref[...] = s + eb_ref[...].reshape(E, 1)
    x16_ref[...] = x_ref[...].astype(jnp.bfloat16)


def _scores(x, gate_weight, e_bias):
    return pl.pallas_call(
        _scores_body,
        out_shape=(jax.ShapeDtypeStruct((E, T), jnp.float32),
                   jax.ShapeDtypeStruct((E, T), jnp.float32),
                   jax.ShapeDtypeStruct((T, DIM), jnp.bfloat16)),
    )(x, gate_weight, e_bias.reshape(E, 1))


TPW = T // NTILE


def _topk_body(sct_hbm, bit_hbm, e1_hbm, e2_hbm, w1_hbm, w2_hbm,
               s_vm, b_vm, e1_vm, e2_vm, w1_vm, w2_vm):
    cid = lax.axis_index("c")
    sid = lax.axis_index("s")

    @pl.when(cid == 0)
    def _():
        _topk_tile(sid, sct_hbm, bit_hbm, e1_hbm, e2_hbm, w1_hbm, w2_hbm,
                   s_vm, b_vm, e1_vm, e2_vm, w1_vm, w2_vm)


def _topk_tile(sid, sct_hbm, bit_hbm, e1_hbm, e2_hbm, w1_hbm, w2_hbm,
               s_vm, b_vm, e1_vm, e2_vm, w1_vm, w2_vm):
    base = sid * TPW
    lane = lax.iota(jnp.int32, 16)
    pltpu.sync_copy(sct_hbm.at[:, pl.ds(base, TPW)], s_vm)
    pltpu.sync_copy(bit_hbm.at[:, pl.ds(base, TPW)], b_vm)
    for g in range(TPW // 16):
        bv = [b_vm[e, pl.ds(g * 16, 16)] for e in range(E)]
        m1 = bv[0]
        for e in range(1, E):
            m1 = jnp.maximum(m1, bv[e])
        i1 = jnp.full((16,), E, jnp.int32)
        for e in range(E):
            i1 = jnp.minimum(i1, jnp.where(bv[e] == m1,
                                           jnp.int32(e), jnp.int32(E)))
        ninf = jnp.float32(-jnp.inf)
        b2 = [jnp.where(i1 == e, ninf, bv[e]) for e in range(E)]
        m2 = b2[0]
        for e in range(1, E):
            m2 = jnp.maximum(m2, b2[e])
        i2 = jnp.full((16,), E, jnp.int32)
        for e in range(E):
            i2 = jnp.minimum(i2, jnp.where(b2[e] == m2,
                                           jnp.int32(e), jnp.int32(E)))
        s1 = jnp.zeros((16,), jnp.float32)
        s2 = jnp.zeros((16,), jnp.float32)
        for e in range(E):
            sv = s_vm[e, pl.ds(g * 16, 16)]
            s1 = jnp.where(i1 == e, sv, s1)
            s2 = jnp.where(i2 == e, sv, s2)
        nrm = jnp.float32(ROUTE_SCALE) / (s1 + s2)
        e1_vm[pl.ds(g * 16, 16)] = i1
        e2_vm[pl.ds(g * 16, 16)] = i2
        w1_vm[pl.ds(g * 16, 16)] = s1 * nrm
        w2_vm[pl.ds(g * 16, 16)] = s2 * nrm
    pltpu.sync_copy(e1_vm, e1_hbm.at[pl.ds(base, TPW)])
    pltpu.sync_copy(e2_vm, e2_hbm.at[pl.ds(base, TPW)])
    pltpu.sync_copy(w1_vm, w1_hbm.at[pl.ds(base, TPW)])
    pltpu.sync_copy(w2_vm, w2_hbm.at[pl.ds(base, TPW)])


def _topk(scoresT, biasedT):
    return pl.kernel(
        _topk_body,
        out_type=(jax.ShapeDtypeStruct((T,), jnp.int32),
                  jax.ShapeDtypeStruct((T,), jnp.int32),
                  jax.ShapeDtypeStruct((T,), jnp.float32),
                  jax.ShapeDtypeStruct((T,), jnp.float32)),
        mesh=_MESH,
        compiler_params=pltpu.CompilerParams(needs_layout_passes=False),
        scratch_types=[
            pltpu.VMEM((E, TPW), jnp.float32),
            pltpu.VMEM((E, TPW), jnp.float32),
            pltpu.VMEM((TPW,), jnp.int32),
            pltpu.VMEM((TPW,), jnp.int32),
            pltpu.VMEM((TPW,), jnp.float32),
            pltpu.VMEM((TPW,), jnp.float32),
        ],
    )(scoresT, biasedT)


def _sort_body(e1_hbm, e2_hbm, w1_hbm, w2_hbm, tok_hbm, ws_hbm, be_hbm,
               pos_hbm, e1_vm, e2_vm, w1f_vm, w2f_vm, tok_vm, ws_vm,
               pa_vm, pb_vm, be_vm):
    cid = lax.axis_index("c")
    sid = lax.axis_index("s")

    @pl.when(jnp.logical_and(cid == 0, sid == 0))
    def _():
        lane = lax.iota(jnp.int32, 16)
        pltpu.sync_copy(e1_hbm, e1_vm)
        pltpu.sync_copy(e2_hbm, e2_vm)
        pltpu.sync_copy(w1_hbm, w1f_vm)
        pltpu.sync_copy(w2_hbm, w2f_vm)

        def zero_body(i, _):
            tok_vm[pl.ds(i * 16, 16)] = jnp.zeros((16,), jnp.int32)
            return 0

        lax.fori_loop(0, R // 16, zero_body, 0)

        def hist_body(g, cnt):
            ev1 = e1_vm[pl.ds(g * 16, 16)]
            ev2 = e2_vm[pl.ds(g * 16, 16)]
            for e in range(E):
                n_e = (jnp.sum(jnp.where(ev1 == e, 1, 0))
                       + jnp.sum(jnp.where(ev2 == e, 1, 0)))
                cnt = cnt + jnp.where(lane == e, n_e, 0)
            return cnt

        cnt = lax.fori_loop(0, T // 16, hist_body, jnp.zeros((16,), jnp.int32))
        nb = (cnt + (BLK - 1)) // BLK
        cum = plsc.cumsum(nb)
        start = (cum - nb) * BLK

        for c in range(2):
            bvv = lane + c * 16
            acc = jnp.zeros((16,), jnp.int32)
            for e in range(E):
                ce = jnp.sum(jnp.where(lane == e, cum, 0))
                acc = acc + jnp.where(bvv >= ce, 1, 0)
            acc = jnp.where(acc >= E, 0, acc)
            be_vm[pl.ds(c * 16, 16)] = acc
        pltpu.sync_copy(be_vm, be_hbm)

        def pos_body(g, nxt):
            tokv = g * 16 + lane
            ev1 = e1_vm[pl.ds(g * 16, 16)]
            ev2 = e2_vm[pl.ds(g * 16, 16)]
            wv1 = w1f_vm[pl.ds(g * 16, 16)]
            wv2 = w2f_vm[pl.ds(g * 16, 16)]
            out = []
            for ev in (ev1, ev2):
                pos = jnp.zeros((16,), jnp.int32)
                for e in range(E):
                    m = ev == e
                    mi = jnp.where(m, 1, 0)
                    rank = plsc.cumsum(mi)
                    ne = jnp.sum(jnp.where(lane == e, nxt, 0))
                    pos = jnp.where(m, ne + rank - 1, pos)
                    nxt = nxt + jnp.where(lane == e, jnp.sum(mi), 0)
                out.append(pos)
            plsc.store_scatter(tok_vm, [out[0]], tokv)
            plsc.store_scatter(tok_vm, [out[1]], tokv)
            plsc.store_scatter(ws_vm, [out[0]], wv1)
            plsc.store_scatter(ws_vm, [out[1]], wv2)
            pa_vm[pl.ds(g * 16, 16)] = out[0]
            pb_vm[pl.ds(g * 16, 16)] = out[1]
            return nxt

        lax.fori_loop(0, T // 16, pos_body, start)
        pltpu.sync_copy(tok_vm, tok_hbm)
        pltpu.sync_copy(ws_vm, ws_hbm)
        pltpu.sync_copy(pa_vm, pos_hbm.at[pl.ds(0, T)])
        pltpu.sync_copy(pb_vm, pos_hbm.at[pl.ds(T, T)])


def _sort(e1, e2, w1, w2):
    return pl.kernel(
        _sort_body,
        out_type=(jax.ShapeDtypeStruct((R,), jnp.int32),
                  jax.ShapeDtypeStruct((R,), jnp.float32),
                  jax.ShapeDtypeStruct((32,), jnp.int32),
                  jax.ShapeDtypeStruct((TOPK * T,), jnp.int32)),
        mesh=_MESH,
        compiler_params=pltpu.CompilerParams(needs_layout_passes=False),
        scratch_types=[
            pltpu.VMEM((T,), jnp.int32),
            pltpu.VMEM((T,), jnp.int32),
            pltpu.VMEM((T,), jnp.float32),
            pltpu.VMEM((T,), jnp.float32),
            pltpu.VMEM((R,), jnp.int32),
            pltpu.VMEM((R,), jnp.float32),
            pltpu.VMEM((T,), jnp.int32),
            pltpu.VMEM((T,), jnp.int32),
            pltpu.VMEM((32,), jnp.int32),
        ],
    )(e1, e2, w1, w2)


_GCH = 96


def _gather_body(x_hbm, tok_hbm, xs_hbm, idx_vm, rows0_vm, rows1_vm,
                 sem0, sem1):
    wid = lax.axis_index("s") * 2 + lax.axis_index("c")
    rows = R // NW
    rb = wid * rows
    pltpu.sync_copy(tok_hbm.at[wid], idx_vm)
    cp0 = pltpu.async_copy(x_hbm.at[idx_vm.at[0]], rows0_vm, sem0)
    cp1 = pltpu.async_copy(x_hbm.at[idx_vm.at[1]], rows1_vm, sem1)
    cp0.wait()
    pltpu.sync_copy(rows0_vm, xs_hbm.at[pl.ds(rb, _GCH)])
    cp1.wait()
    pltpu.sync_copy(rows1_vm, xs_hbm.at[pl.ds(rb + _GCH, _GCH)])


def _gather(x16i, tok_sorted):
    return pl.kernel(
        _gather_body,
        out_type=jax.ShapeDtypeStruct((R, DIM // 2), jnp.int32),
        mesh=_MESH,
        compiler_params=pltpu.CompilerParams(needs_layout_passes=False),
        scratch_types=[
            pltpu.VMEM((2, _GCH), jnp.int32),
            pltpu.VMEM((_GCH, DIM // 2), jnp.int32),
            pltpu.VMEM((_GCH, DIM // 2), jnp.int32),
            pltpu.SemaphoreType.DMA,
            pltpu.SemaphoreType.DMA,
        ],
    )(x16i, tok_sorted.reshape(NW, 2, _GCH))


def _gffn_body(be_ref, tok_ref, x16_ref, gp_ref, up_ref, dp_ref, ws_ref,
               y_ref):
    tokv = tok_ref[0, 0]
    iota_t = lax.broadcasted_iota(jnp.int32, (BLK, T), 1)
    pmat = (iota_t == tokv.reshape(BLK, 1)).astype(jnp.bfloat16)
    xb16 = lax.dot_general(pmat, x16_ref[...], (((1,), (0,)), ((), ())),
                           preferred_element_type=jnp.float32
                           ).astype(jnp.bfloat16)
    g = lax.dot_general(xb16, gp_ref[0], (((1,), (1,)), ((), ())),
                        preferred_element_type=jnp.float32)
    u = lax.dot_general(xb16, up_ref[0], (((1,), (1,)), ((), ())),
                        preferred_element_type=jnp.float32)
    inter = (jax.nn.silu(g) * u).astype(jnp.bfloat16)
    h = lax.dot_general(inter, dp_ref[0], (((1,), (1,)), ((), ())),
                        preferred_element_type=jnp.float32)
    y_ref[...] = ws_ref[...] * h


def _gffn(block_expert, tok_sorted, x16, gp16, up16, dp16, ws):
    return pl.pallas_call(
        _gffn_body,
        grid_spec=pltpu.PrefetchScalarGridSpec(
            num_scalar_prefetch=1,
            grid=(NBLK2,),
            in_specs=[
                pl.BlockSpec((1, 1, BLK), lambda b, be: (b, 0, 0)),
                pl.BlockSpec((T, DIM), lambda b, be: (0, 0)),
                pl.BlockSpec((1, INTER, DIM), lambda b, be: (be[b], 0, 0)),
                pl.BlockSpec((1, INTER, DIM), lambda b, be: (be[b], 0, 0)),
                pl.BlockSpec((1, DIM, INTER), lambda b, be: (be[b], 0, 0)),
                pl.BlockSpec((BLK, 1), lambda b, be: (b, 0)),
            ],
            out_specs=pl.BlockSpec((BLK, DIM), lambda b, be: (b, 0)),
        ),
        out_shape=jax.ShapeDtypeStruct((R2, DIM), jnp.float32),
        compiler_params=pltpu.CompilerParams(
            dimension_semantics=("arbitrary",)),
    )(block_expert, tok_sorted, x16, gp16, up16, dp16, ws)


def _shared_body(x_ref, gp_ref, up_ref, dp_ref, z_ref):
    xb16 = x_ref[...].astype(jnp.bfloat16)
    g = lax.dot_general(xb16, gp_ref[...], (((1,), (1,)), ((), ())),
                        preferred_element_type=jnp.float32)
    u = lax.dot_general(xb16, up_ref[...], (((1,), (1,)), ((), ())),
                        preferred_element_type=jnp.float32)
    inter = (jax.nn.silu(g) * u).astype(jnp.bfloat16)
    z_ref[...] = lax.dot_general(inter, dp_ref[...], (((1,), (1,)), ((), ())),
                                 preferred_element_type=jnp.float32)


def _shared(x, sg16, su16, sd16):
    bt = 512
    return pl.pallas_call(
        _shared_body,
        grid=(T // bt,),
        in_specs=[
            pl.BlockSpec((bt, DIM), lambda i: (i, 0)),
            pl.BlockSpec((INTER, DIM), lambda i: (0, 0)),
            pl.BlockSpec((INTER, DIM), lambda i: (0, 0)),
            pl.BlockSpec((DIM, INTER), lambda i: (0, 0)),
        ],
        out_specs=pl.BlockSpec((bt, DIM), lambda i: (i, 0)),
        out_shape=jax.ShapeDtypeStruct((T, DIM), jnp.float32),
        compiler_params=pltpu.CompilerParams(
            dimension_semantics=("parallel",)),
    )(x, sg16, su16, sd16)


_CCH = 16


def _combine_body(y_hbm, pos_hbm, out_hbm,
                  p0_vm, p1_vm, y0_vm, y1_vm, z_vm, sem):
    wid = lax.axis_index("s") * 2 + lax.axis_index("c")
    tpw = T // NW
    for c in range(tpw // _CCH):
        tb = wid * tpw + c * _CCH
        pltpu.sync_copy(pos_hbm.at[pl.ds(tb, _CCH)], p0_vm)
        pltpu.sync_copy(pos_hbm.at[pl.ds(T + tb, _CCH)], p1_vm)
        cp0 = pltpu.async_copy(y_hbm.at[p0_vm], y0_vm, sem)
        cp1 = pltpu.async_copy(y_hbm.at[p1_vm], y1_vm, sem)
        pltpu.sync_copy(y_hbm.at[pl.ds(R + tb, _CCH)], z_vm)
        cp0.wait()
        cp1.wait()

        def body(r, _):
            for u in range(DIM // 16):
                col = u * 16
                z_vm[r, pl.ds(col, 16)] = (z_vm[r, pl.ds(col, 16)]
                                           + y0_vm[r, pl.ds(col, 16)]
                                           + y1_vm[r, pl.ds(col, 16)])
            return 0

        lax.fori_loop(0, _CCH, body, 0)
        pltpu.sync_copy(z_vm, out_hbm.at[pl.ds(tb, _CCH)])


def _combine(y_sorted, pos2):
    return pl.kernel(
        _combine_body,
        out_type=jax.ShapeDtypeStruct((T, DIM), jnp.float32),
        mesh=_MESH,
        compiler_params=pltpu.CompilerParams(needs_layout_passes=False),
        scratch_types=[
            pltpu.VMEM((_CCH,), jnp.int32),
            pltpu.VMEM((_CCH,), jnp.int32),
            pltpu.VMEM((_CCH, DIM), jnp.float32),
            pltpu.VMEM((_CCH, DIM), jnp.float32),
            pltpu.VMEM((_CCH, DIM), jnp.float32),
            pltpu.SemaphoreType.DMA,
        ],
    )(y_sorted, pos2)


@jax.jit
def _moe(x, gate_weight, e_bias, gp16, up16, dp16):
    scoresT, biasedT, x16 = _scores(x, gate_weight, e_bias)
    e1, e2, w1, w2 = _topk(scoresT, biasedT)
    tok_sorted, w_sorted, block_expert, pos2 = _sort(e1, e2, w1, w2)
    tok_ext = jnp.concatenate(
        [tok_sorted, jnp.arange(T, dtype=jnp.int32)]).reshape(NBLK2, 1, BLK)
    be_ext = jnp.concatenate(
        [block_expert[:NBLK],
         jnp.full((NBLK2 - NBLK,), E, jnp.int32)])
    ws_ext = jnp.concatenate(
        [w_sorted, jnp.ones((T,), jnp.float32)]).reshape(R2, 1)
    y = _gffn(be_ext, tok_ext, x16, gp16, up16, dp16, ws_ext)
    return _combine(y, pos2)


def kernel(x, token_mask, gate_weight, e_bias, gate_projs, up_projs,
           down_projs, shared_gate, shared_up, shared_down):
    del token_mask
    gp16 = jnp.concatenate([gate_projs, shared_gate[None]],
                           axis=0).astype(jnp.bfloat16)
    up16 = jnp.concatenate([up_projs, shared_up[None]],
                           axis=0).astype(jnp.bfloat16)
    dp16 = jnp.concatenate([down_projs, shared_down[None]],
                           axis=0).astype(jnp.bfloat16)
    return _moe(x, gate_weight, e_bias, gp16, up16, dp16)


# speedup vs baseline: 1.2354x; 1.2354x over previous
import functools

import jax
import jax.numpy as jnp
from jax import lax
from jax.experimental import pallas as pl
from jax.experimental.pallas import tpu as pltpu
from jax.experimental.pallas import tpu_sc as plsc

E = 8
TOPK = 2
DIM = 1024
INTER = 512
ROUTE_SCALE = 2.5
T = 2048

BLK = 256
NBLK = T * TOPK // BLK + E
R = NBLK * BLK
NBLK2 = NBLK + T // BLK
R2 = NBLK2 * BLK

NTILE = 16
NW = 32
TT = T // NTILE
_MESH = plsc.VectorSubcoreMesh(core_axis_name="c", subcore_axis_name="s",
                               num_cores=2, num_subcores=NTILE)


def _scores_body(x_ref, gw_ref, eb_ref, sc_ref, bi_ref, x16_ref):
    st = lax.dot_general(gw_ref[...], x_ref[...], (((1,), (1,)), ((), ())),
                         preferred_element_type=jnp.float32)
    s = jax.nn.sigmoid(st)
    sc_ref[...] = s
    bi_ref[...] = s + eb_ref[...].reshape(E, 1)
    x16_ref[...] = x_ref[...].astype(jnp.bfloat16)


def _scores(x, gate_weight, e_bias):
    return pl.pallas_call(
        _scores_body,
        out_shape=(jax.ShapeDtypeStruct((E, T), jnp.float32),
                   jax.ShapeDtypeStruct((E, T), jnp.float32),
                   jax.ShapeDtypeStruct((T, DIM), jnp.bfloat16)),
    )(x, gate_weight, e_bias.reshape(E, 1))


TPW = T // NTILE


def _topk_body(sct_hbm, bit_hbm, e1_hbm, e2_hbm, w1_hbm, w2_hbm,
               s_vm, b_vm, e1_vm, e2_vm, w1_vm, w2_vm):
    cid = lax.axis_index("c")
    sid = lax.axis_index("s")

    @pl.when(cid == 0)
    def _():
        _topk_tile(sid, sct_hbm, bit_hbm, e1_hbm, e2_hbm, w1_hbm, w2_hbm,
                   s_vm, b_vm, e1_vm, e2_vm, w1_vm, w2_vm)


def _topk_tile(sid, sct_hbm, bit_hbm, e1_hbm, e2_hbm, w1_hbm, w2_hbm,
               s_vm, b_vm, e1_vm, e2_vm, w1_vm, w2_vm):
    base = sid * TPW
    lane = lax.iota(jnp.int32, 16)
    pltpu.sync_copy(sct_hbm.at[:, pl.ds(base, TPW)], s_vm)
    pltpu.sync_copy(bit_hbm.at[:, pl.ds(base, TPW)], b_vm)
    for g in range(TPW // 16):
        bv = [b_vm[e, pl.ds(g * 16, 16)] for e in range(E)]
        m1 = bv[0]
        for e in range(1, E):
            m1 = jnp.maximum(m1, bv[e])
        i1 = jnp.full((16,), E, jnp.int32)
        for e in range(E):
            i1 = jnp.minimum(i1, jnp.where(bv[e] == m1,
                                           jnp.int32(e), jnp.int32(E)))
        ninf = jnp.float32(-jnp.inf)
        b2 = [jnp.where(i1 == e, ninf, bv[e]) for e in range(E)]
        m2 = b2[0]
        for e in range(1, E):
            m2 = jnp.maximum(m2, b2[e])
        i2 = jnp.full((16,), E, jnp.int32)
        for e in range(E):
            i2 = jnp.minimum(i2, jnp.where(b2[e] == m2,
                                           jnp.int32(e), jnp.int32(E)))
        s1 = jnp.zeros((16,), jnp.float32)
        s2 = jnp.zeros((16,), jnp.float32)
        for e in range(E):
            sv = s_vm[e, pl.ds(g * 16, 16)]
            s1 = jnp.where(i1 == e, sv, s1)
            s2 = jnp.where(i2 == e, sv, s2)
        nrm = jnp.float32(ROUTE_SCALE) / (s1 + s2)
        e1_vm[pl.ds(g * 16, 16)] = i1
        e2_vm[pl.ds(g * 16, 16)] = i2
        w1_vm[pl.ds(g * 16, 16)] = s1 * nrm
        w2_vm[pl.ds(g * 16, 16)] = s2 * nrm
    pltpu.sync_copy(e1_vm, e1_hbm.at[pl.ds(base, TPW)])
    pltpu.sync_copy(e2_vm, e2_hbm.at[pl.ds(base, TPW)])
    pltpu.sync_copy(w1_vm, w1_hbm.at[pl.ds(base, TPW)])
    pltpu.sync_copy(w2_vm, w2_hbm.at[pl.ds(base, TPW)])


def _topk(scoresT, biasedT):
    return pl.kernel(
        _topk_body,
        out_type=(jax.ShapeDtypeStruct((T,), jnp.int32),
                  jax.ShapeDtypeStruct((T,), jnp.int32),
                  jax.ShapeDtypeStruct((T,), jnp.float32),
                  jax.ShapeDtypeStruct((T,), jnp.float32)),
        mesh=_MESH,
        compiler_params=pltpu.CompilerParams(needs_layout_passes=False),
        scratch_types=[
            pltpu.VMEM((E, TPW), jnp.float32),
            pltpu.VMEM((E, TPW), jnp.float32),
            pltpu.VMEM((TPW,), jnp.int32),
            pltpu.VMEM((TPW,), jnp.int32),
            pltpu.VMEM((TPW,), jnp.float32),
            pltpu.VMEM((TPW,), jnp.float32),
        ],
    )(scoresT, biasedT)


def _sort_body(e1_hbm, e2_hbm, w1_hbm, w2_hbm, tok_hbm, ws_hbm, be_hbm,
               pos_hbm, e1_vm, e2_vm, w1f_vm, w2f_vm, tok_vm, ws_vm,
               pa_vm, pb_vm, be_vm):
    cid = lax.axis_index("c")
    sid = lax.axis_index("s")

    @pl.when(jnp.logical_and(cid == 0, sid == 0))
    def _():
        lane = lax.iota(jnp.int32, 16)
        pltpu.sync_copy(e1_hbm, e1_vm)
        pltpu.sync_copy(e2_hbm, e2_vm)
        pltpu.sync_copy(w1_hbm, w1f_vm)
        pltpu.sync_copy(w2_hbm, w2f_vm)

        def zero_body(i, _):
            tok_vm[pl.ds(i * 16, 16)] = jnp.zeros((16,), jnp.int32)
            return 0

        lax.fori_loop(0, R // 16, zero_body, 0)

        def hist_body(g4, cnt):
            for gg in range(4):
                go = (g4 * 4 + gg) * 16
                ev1 = e1_vm[pl.ds(go, 16)]
                ev2 = e2_vm[pl.ds(go, 16)]
                for e in range(E):
                    n_e = (jnp.sum(jnp.where(ev1 == e, 1, 0))
                           + jnp.sum(jnp.where(ev2 == e, 1, 0)))
                    cnt = cnt + jnp.where(lane == e, n_e, 0)
            return cnt

        cnt = lax.fori_loop(0, T // 64, hist_body, jnp.zeros((16,), jnp.int32))
        nb = (cnt + (BLK - 1)) // BLK
        cum = plsc.cumsum(nb)
        start = (cum - nb) * BLK

        for c in range(2):
            bvv = lane + c * 16
            acc = jnp.zeros((16,), jnp.int32)
            for e in range(E):
                ce = jnp.sum(jnp.where(lane == e, cum, 0))
                acc = acc + jnp.where(bvv >= ce, 1, 0)
            acc = jnp.where(acc >= E, 0, acc)
            be_vm[pl.ds(c * 16, 16)] = acc
        pltpu.sync_copy(be_vm, be_hbm)

        def pos_body(g2, nxt):
            for gg in range(2):
                go = (g2 * 2 + gg) * 16
                tokv = go + lane
                ev1 = e1_vm[pl.ds(go, 16)]
                ev2 = e2_vm[pl.ds(go, 16)]
                wv1 = w1f_vm[pl.ds(go, 16)]
                wv2 = w2f_vm[pl.ds(go, 16)]
                out = []
                for ev in (ev1, ev2):
                    pos = jnp.zeros((16,), jnp.int32)
                    for e in range(E):
                        m = ev == e
                        mi = jnp.where(m, 1, 0)
                        rank = plsc.cumsum(mi)
                        ne = jnp.sum(jnp.where(lane == e, nxt, 0))
                        pos = jnp.where(m, ne + rank - 1, pos)
                        nxt = nxt + jnp.where(lane == e, jnp.sum(mi), 0)
                    out.append(pos)
                plsc.store_scatter(tok_vm, [out[0]], tokv)
                plsc.store_scatter(tok_vm, [out[1]], tokv)
                plsc.store_scatter(ws_vm, [out[0]], wv1)
                plsc.store_scatter(ws_vm, [out[1]], wv2)
                pa_vm[pl.ds(go, 16)] = out[0]
                pb_vm[pl.ds(go, 16)] = out[1]
            return nxt

        lax.fori_loop(0, T // 32, pos_body, start)
        pltpu.sync_copy(tok_vm, tok_hbm)
        pltpu.sync_copy(ws_vm, ws_hbm)
        pltpu.sync_copy(pa_vm, pos_hbm.at[pl.ds(0, T)])
        pltpu.sync_copy(pb_vm, pos_hbm.at[pl.ds(T, T)])


def _sort(e1, e2, w1, w2):
    return pl.kernel(
        _sort_body,
        out_type=(jax.ShapeDtypeStruct((R,), jnp.int32),
                  jax.ShapeDtypeStruct((R,), jnp.float32),
                  jax.ShapeDtypeStruct((32,), jnp.int32),
                  jax.ShapeDtypeStruct((TOPK * T,), jnp.int32)),
        mesh=_MESH,
        compiler_params=pltpu.CompilerParams(needs_layout_passes=False),
        scratch_types=[
            pltpu.VMEM((T,), jnp.int32),
            pltpu.VMEM((T,), jnp.int32),
            pltpu.VMEM((T,), jnp.float32),
            pltpu.VMEM((T,), jnp.float32),
            pltpu.VMEM((R,), jnp.int32),
            pltpu.VMEM((R,), jnp.float32),
            pltpu.VMEM((T,), jnp.int32),
            pltpu.VMEM((T,), jnp.int32),
            pltpu.VMEM((32,), jnp.int32),
        ],
    )(e1, e2, w1, w2)


_GCH = 96


def _gather_body(x_hbm, tok_hbm, xs_hbm, idx_vm, rows0_vm, rows1_vm,
                 sem0, sem1):
    wid = lax.axis_index("s") * 2 + lax.axis_index("c")
    rows = R // NW
    rb = wid * rows
    pltpu.sync_copy(tok_hbm.at[wid], idx_vm)
    cp0 = pltpu.async_copy(x_hbm.at[idx_vm.at[0]], rows0_vm, sem0)
    cp1 = pltpu.async_copy(x_hbm.at[idx_vm.at[1]], rows1_vm, sem1)
    cp0.wait()
    pltpu.sync_copy(rows0_vm, xs_hbm.at[pl.ds(rb, _GCH)])
    cp1.wait()
    pltpu.sync_copy(rows1_vm, xs_hbm.at[pl.ds(rb + _GCH, _GCH)])


def _gather(x16i, tok_sorted):
    return pl.kernel(
        _gather_body,
        out_type=jax.ShapeDtypeStruct((R, DIM // 2), jnp.int32),
        mesh=_MESH,
        compiler_params=pltpu.CompilerParams(needs_layout_passes=False),
        scratch_types=[
            pltpu.VMEM((2, _GCH), jnp.int32),
            pltpu.VMEM((_GCH, DIM // 2), jnp.int32),
            pltpu.VMEM((_GCH, DIM // 2), jnp.int32),
            pltpu.SemaphoreType.DMA,
            pltpu.SemaphoreType.DMA,
        ],
    )(x16i, tok_sorted.reshape(NW, 2, _GCH))


def _gffn_body(be_ref, tok_ref, x16_ref, gp_ref, up_ref, dp_ref, ws_ref,
               y_ref):
    tokv = tok_ref[0, 0]
    iota_t = lax.broadcasted_iota(jnp.int32, (BLK, T), 1)
    pmat = (iota_t == tokv.reshape(BLK, 1)).astype(jnp.bfloat16)
    xb16 = lax.dot_general(pmat, x16_ref[...], (((1,), (0,)), ((), ())),
                           preferred_element_type=jnp.float32
                           ).astype(jnp.bfloat16)
    g = lax.dot_general(xb16, gp_ref[0], (((1,), (1,)), ((), ())),
                        preferred_element_type=jnp.float32)
    u = lax.dot_general(xb16, up_ref[0], (((1,), (1,)), ((), ())),
                        preferred_element_type=jnp.float32)
    inter = (jax.nn.silu(g) * u).astype(jnp.bfloat16)
    h = lax.dot_general(inter, dp_ref[0], (((1,), (1,)), ((), ())),
                        preferred_element_type=jnp.float32)
    y_ref[...] = ws_ref[...] * h


def _gffn(block_expert, tok_sorted, x16, gp16, up16, dp16, ws):
    return pl.pallas_call(
        _gffn_body,
        grid_spec=pltpu.PrefetchScalarGridSpec(
            num_scalar_prefetch=1,
            grid=(NBLK,),
            in_specs=[
                pl.BlockSpec((1, 1, BLK), lambda b, be: (b, 0, 0)),
                pl.BlockSpec((T, DIM), lambda b, be: (0, 0)),
                pl.BlockSpec((1, INTER, DIM), lambda b, be: (be[b], 0, 0)),
                pl.BlockSpec((1, INTER, DIM), lambda b, be: (be[b], 0, 0)),
                pl.BlockSpec((1, DIM, INTER), lambda b, be: (be[b], 0, 0)),
                pl.BlockSpec((BLK, 1), lambda b, be: (b, 0)),
            ],
            out_specs=pl.BlockSpec((BLK, DIM), lambda b, be: (b, 0)),
        ),
        out_shape=jax.ShapeDtypeStruct((R, DIM), jnp.float32),
        compiler_params=pltpu.CompilerParams(
            dimension_semantics=("arbitrary",)),
    )(block_expert, tok_sorted, x16, gp16, up16, dp16, ws)


def _shared_body(x_ref, gp_ref, up_ref, dp_ref, z_ref):
    xb16 = x_ref[...].astype(jnp.bfloat16)
    g = lax.dot_general(xb16, gp_ref[...], (((1,), (1,)), ((), ())),
                        preferred_element_type=jnp.float32)
    u = lax.dot_general(xb16, up_ref[...], (((1,), (1,)), ((), ())),
                        preferred_element_type=jnp.float32)
    inter = (jax.nn.silu(g) * u).astype(jnp.bfloat16)
    z_ref[...] = lax.dot_general(inter, dp_ref[...], (((1,), (1,)), ((), ())),
                                 preferred_element_type=jnp.float32)


def _shared(x, sg16, su16, sd16):
    bt = 512
    return pl.pallas_call(
        _shared_body,
        grid=(T // bt,),
        in_specs=[
            pl.BlockSpec((bt, DIM), lambda i: (i, 0)),
            pl.BlockSpec((INTER, DIM), lambda i: (0, 0)),
            pl.BlockSpec((INTER, DIM), lambda i: (0, 0)),
            pl.BlockSpec((DIM, INTER), lambda i: (0, 0)),
        ],
        out_specs=pl.BlockSpec((bt, DIM), lambda i: (i, 0)),
        out_shape=jax.ShapeDtypeStruct((T, DIM), jnp.float32),
        compiler_params=pltpu.CompilerParams(
            dimension_semantics=("parallel",)),
    )(x, sg16, su16, sd16)


_CCH = 16


def _combine_body(y_hbm, z_hbm, pos_hbm, out_hbm,
                  p_vm, y0_vm, y1_vm, z_vm, sems):
    wid = lax.axis_index("s") * 2 + lax.axis_index("c")
    tpw = T // NW
    nch = tpw // _CCH
    base = wid * tpw
    pltpu.sync_copy(pos_hbm.at[pl.ds(base, tpw)], p_vm.at[0])
    pltpu.sync_copy(pos_hbm.at[pl.ds(T + base, tpw)], p_vm.at[1])

    def fire(c, buf):
        off = c * _CCH
        cp0 = pltpu.async_copy(y_hbm.at[p_vm.at[0, pl.ds(off, _CCH)]],
                               y0_vm.at[buf], sems[buf])
        cp1 = pltpu.async_copy(y_hbm.at[p_vm.at[1, pl.ds(off, _CCH)]],
                               y1_vm.at[buf], sems[buf])
        cpz = pltpu.async_copy(z_hbm.at[pl.ds(base + off, _CCH)],
                               z_vm.at[buf], sems[buf])
        return (cp0, cp1, cpz)

    pend = fire(0, 0)
    for c in range(nch):
        buf = c % 2
        for cp in pend:
            cp.wait()
        if c + 1 < nch:
            pend = fire(c + 1, 1 - buf)

        def body(r, _, buf=buf):
            for u in range(DIM // 16):
                col = u * 16
                z_vm[buf, r, pl.ds(col, 16)] = (
                    z_vm[buf, r, pl.ds(col, 16)]
                    + y0_vm[buf, r, pl.ds(col, 16)]
                    + y1_vm[buf, r, pl.ds(col, 16)])
            return 0

        lax.fori_loop(0, _CCH, body, 0)
        pltpu.sync_copy(z_vm.at[buf], out_hbm.at[pl.ds(base + c * _CCH, _CCH)])


def _combine(y_sorted, z, pos2):
    return pl.kernel(
        _combine_body,
        out_type=jax.ShapeDtypeStruct((T, DIM), jnp.float32),
        mesh=_MESH,
        compiler_params=pltpu.CompilerParams(needs_layout_passes=False),
        scratch_types=[
            pltpu.VMEM((2, T // NW), jnp.int32),
            pltpu.VMEM((2, _CCH, DIM), jnp.float32),
            pltpu.VMEM((2, _CCH, DIM), jnp.float32),
            pltpu.VMEM((2, _CCH, DIM), jnp.float32),
            [pltpu.SemaphoreType.DMA, pltpu.SemaphoreType.DMA],
        ],
    )(y_sorted, z, pos2)


@jax.jit
def _moe(x, gate_weight, e_bias, gp16, up16, dp16, sg16, su16, sd16):
    scoresT, biasedT, x16 = _scores(x, gate_weight, e_bias)
    e1, e2, w1, w2 = _topk(scoresT, biasedT)
    tok_sorted, w_sorted, block_expert, pos2 = _sort(e1, e2, w1, w2)
    y = _gffn(block_expert[:NBLK], tok_sorted.reshape(NBLK, 1, BLK), x16,
              gp16, up16, dp16, w_sorted.reshape(R, 1))
    z = _shared(x, sg16, su16, sd16)
    return _combine(y, z, pos2)


def kernel(x, token_mask, gate_weight, e_bias, gate_projs, up_projs,
           down_projs, shared_gate, shared_up, shared_down):
    del token_mask
    gp16 = gate_projs.astype(jnp.bfloat16)
    up16 = up_projs.astype(jnp.bfloat16)
    dp16 = down_projs.astype(jnp.bfloat16)
    sg16 = shared_gate.astype(jnp.bfloat16)
    su16 = shared_up.astype(jnp.bfloat16)
    sd16 = shared_down.astype(jnp.bfloat16)
    return _moe(x, gate_weight, e_bias, gp16, up16, dp16, sg16, su16, sd16)
